# baseline (speedup 1.0000x reference)
import jax
import jax.numpy as jnp
from jax.experimental import pallas as pl
from jax.experimental.pallas import tpu as pltpu

EMB = 64


def _body(b_ref, tflat_ref, out_ref, p_ref):
    L = b_ref.shape[1]
    N = L * EMB

    @pl.when(pl.program_id(0) == 0)
    def _init():
        row = jax.lax.broadcasted_iota(jnp.int32, (L, N), 0)
        col = jax.lax.broadcasted_iota(jnp.int32, (L, N), 1)
        p_ref[...] = (row == col // EMB).astype(jnp.bfloat16)

    mask = (b_ref[...] != 0).astype(jnp.bfloat16)
    y = jax.lax.dot_general(
        mask, p_ref[...],
        dimension_numbers=(((1,), (0,)), ((), ())),
        preferred_element_type=jnp.float32,
    )
    out_ref[...] = y * tflat_ref[...]


def kernel(batch, table):
    B, L = batch.shape
    N = L * EMB
    BB = 256

    tflat = table[1:L + 1].reshape(1, N)

    out = pl.pallas_call(
        _body,
        grid=(B // BB,),
        in_specs=[
            pl.BlockSpec((BB, L), lambda i: (i, 0)),
            pl.BlockSpec((1, N), lambda i: (0, 0)),
        ],
        out_specs=pl.BlockSpec((BB, N), lambda i: (i, 0)),
        out_shape=jax.ShapeDtypeStruct((B, N), jnp.float32),
        scratch_shapes=[pltpu.VMEM((L, N), jnp.bfloat16)],
    )(batch, tflat)
    return out.reshape(B, L, EMB)
